# Initial kernel scaffold; baseline (speedup 1.0000x reference)
import jax
import jax.numpy as jnp
from jax.experimental import pallas as pl


def kernel(bev_feature, batch_centers, num_point):
    raise NotImplementedError("write your pallas kernel here")


# speedup vs baseline: 2.5445x; 2.5445x over previous
import functools

import jax
import jax.numpy as jnp
from jax import lax
from jax.experimental import pallas as pl
from jax.experimental.pallas import tpu as pltpu
from jax.experimental.pallas import tpu_sc as plsc

PC_START = (-54.0, -54.0)
VOXEL_SIZE = (0.075, 0.075)
OUT_STRIDE = 8

B = 4
H = 180
W = 180
C = 512
N = 2500
NSEC = 5
SEC = N // NSEC
NPTS = B * N

NC = 2
NS = 16
NW = NC * NS
LANES = 16
PPW = 320
PPR = 32
RPW = PPW // PPR


def _sc_interp(bev2d, xs, ys, rowbase):
    mesh = plsc.VectorSubcoreMesh(core_axis_name="c", subcore_axis_name="s")

    @functools.partial(
        pl.kernel,
        mesh=mesh,
        out_type=jax.ShapeDtypeStruct((NPTS, C), jnp.float32),
        scratch_types=[
            pltpu.VMEM((PPW,), jnp.float32),
            pltpu.VMEM((PPW,), jnp.float32),
            pltpu.VMEM((PPW,), jnp.int32),
            pltpu.VMEM((4 * PPR,), jnp.int32),
            pltpu.VMEM((4 * PPR, LANES), jnp.float32),
            pltpu.VMEM((4 * PPR, C), jnp.float32),
            pltpu.VMEM((PPR, C), jnp.float32),
            pltpu.SemaphoreType.DMA,
        ],
    )
    def k(bev_hbm, xs_hbm, ys_hbm, base_hbm, out_hbm,
          xv, yv, bv, idx_v, wts_v, rows_v, out_v, sem):
        wid = lax.axis_index("s") * NC + lax.axis_index("c")
        wbase = jnp.minimum(wid * PPW, NPTS - PPW)
        pltpu.sync_copy(xs_hbm.at[pl.ds(wbase, PPW)], xv)
        pltpu.sync_copy(ys_hbm.at[pl.ds(wbase, PPW)], yv)
        pltpu.sync_copy(base_hbm.at[pl.ds(wbase, PPW)], bv)


        def round_body(r, _):
            for t in range(PPR // LANES):
                o = r * PPR + t * LANES
                x = xv[pl.ds(o, LANES)]
                y = yv[pl.ds(o, LANES)]
                bb = bv[pl.ds(o, LANES)]
                x = (x - PC_START[0]) / VOXEL_SIZE[0] / OUT_STRIDE
                y = (y - PC_START[1]) / VOXEL_SIZE[1] / OUT_STRIDE
                x = jnp.minimum(jnp.maximum(x, -4.0), 184.0)
                y = jnp.minimum(jnp.maximum(y, -4.0), 184.0)
                xi = x.astype(jnp.int32)
                yi = y.astype(jnp.int32)
                x0 = jnp.where(xi.astype(jnp.float32) > x, xi - 1, xi)
                y0 = jnp.where(yi.astype(jnp.float32) > y, yi - 1, yi)
                x0c = jnp.minimum(jnp.maximum(x0, 0), W - 1)
                x1c = jnp.minimum(jnp.maximum(x0 + 1, 0), W - 1)
                y0c = jnp.minimum(jnp.maximum(y0, 0), H - 1)
                y1c = jnp.minimum(jnp.maximum(y0 + 1, 0), H - 1)
                fx0 = x0c.astype(jnp.float32)
                fx1 = x1c.astype(jnp.float32)
                fy0 = y0c.astype(jnp.float32)
                fy1 = y1c.astype(jnp.float32)
                wa = (fx1 - x) * (fy1 - y)
                wb = (fx1 - x) * (y - fy0)
                wc = (x - fx0) * (fy1 - y)
                wd = (x - fx0) * (y - fy0)
                for l in range(LANES):
                    p = t * LANES + l
                    wts_v[0 * PPR + p, :] = jnp.full((LANES,), wa[l])
                    wts_v[1 * PPR + p, :] = jnp.full((LANES,), wb[l])
                    wts_v[2 * PPR + p, :] = jnp.full((LANES,), wc[l])
                    wts_v[3 * PPR + p, :] = jnp.full((LANES,), wd[l])
                row0 = bb + y0c * W
                row1 = bb + y1c * W
                idx_v[pl.ds(0 * PPR + t * LANES, LANES)] = row0 + x0c
                idx_v[pl.ds(1 * PPR + t * LANES, LANES)] = row1 + x0c
                idx_v[pl.ds(2 * PPR + t * LANES, LANES)] = row0 + x1c
                idx_v[pl.ds(3 * PPR + t * LANES, LANES)] = row1 + x1c

            pltpu.async_copy(bev_hbm.at[idx_v], rows_v, sem).wait()

            def blend_body(kk, _):
                w_a = wts_v[0 * PPR + kk, :]
                w_b = wts_v[1 * PPR + kk, :]
                w_c = wts_v[2 * PPR + kk, :]
                w_d = wts_v[3 * PPR + kk, :]
                for j in range(C // LANES):
                    s = pl.ds(j * LANES, LANES)
                    va = rows_v[kk, s]
                    vb = rows_v[PPR + kk, s]
                    vc = rows_v[2 * PPR + kk, s]
                    vd = rows_v[3 * PPR + kk, s]
                    out_v[kk, s] = ((va * w_a + vb * w_b) + vc * w_c) + vd * w_d
                return 0

            lax.fori_loop(0, PPR, blend_body, 0)
            pltpu.sync_copy(out_v, out_hbm.at[pl.ds(wbase + r * PPR, PPR)])
            return 0

        lax.fori_loop(0, RPW, round_body, 0)

    return k(bev2d, xs, ys, rowbase)


def kernel(bev_feature, batch_centers, num_point):
    del num_point
    cx = batch_centers[..., 0]
    cy = batch_centers[..., 1]
    order = (jnp.arange(SEC)[:, None] + SEC * jnp.arange(NSEC)[None, :]).reshape(-1)
    xs = cx[:, order].reshape(-1)
    ys = cy[:, order].reshape(-1)
    rowbase = jnp.repeat(jnp.arange(B, dtype=jnp.int32) * (H * W), N)
    bev2d = bev_feature.reshape(B * H * W, C)
    out = _sc_interp(bev2d, xs, ys, rowbase)
    return out.reshape(B, SEC, NSEC * C)


# speedup vs baseline: 2.7685x; 1.0880x over previous
import functools

import jax
import jax.numpy as jnp
from jax import lax
from jax.experimental import pallas as pl
from jax.experimental.pallas import tpu as pltpu
from jax.experimental.pallas import tpu_sc as plsc

PC_START = (-54.0, -54.0)
VOXEL_SIZE = (0.075, 0.075)
OUT_STRIDE = 8

B = 4
H = 180
W = 180
C = 512
N = 2500
NSEC = 5
SEC = N // NSEC
NPTS = B * N
BHW = B * H * W

NC = 2
NS = 16
LANES = 16
PPW = 320
PPH = 16
NH = PPW // PPH
XPAD = PPW + PPH


def _sc_interp(bev2d, xs, ys, rowbase):
    mesh = plsc.VectorSubcoreMesh(core_axis_name="c", subcore_axis_name="s")

    @functools.partial(
        pl.kernel,
        mesh=mesh,
        out_type=jax.ShapeDtypeStruct((NPTS, C), jnp.float32),
        scratch_types=[
            pltpu.VMEM((XPAD,), jnp.float32),
            pltpu.VMEM((XPAD,), jnp.float32),
            pltpu.VMEM((XPAD,), jnp.int32),
            pltpu.VMEM((4 * PPH,), jnp.int32),
            pltpu.VMEM((4 * PPH,), jnp.int32),
            pltpu.VMEM((4 * PPH, LANES), jnp.float32),
            pltpu.VMEM((4 * PPH, LANES), jnp.float32),
            pltpu.VMEM((4 * PPH, C), jnp.float32),
            pltpu.VMEM((4 * PPH, C), jnp.float32),
            pltpu.VMEM((PPH, C), jnp.float32),
            pltpu.VMEM((PPH, C), jnp.float32),
            pltpu.SemaphoreType.DMA,
            pltpu.SemaphoreType.DMA,
            pltpu.SemaphoreType.DMA,
            pltpu.SemaphoreType.DMA,
        ],
    )
    def k(bev_hbm, xs_hbm, ys_hbm, base_hbm, out_hbm,
          xv, yv, bv, idx_a, idx_b, wts_a, wts_b, rows_a, rows_b,
          out_a, out_b, gs_a, gs_b, os_a, os_b):
        wid = lax.axis_index("s") * NC + lax.axis_index("c")
        wbase = jnp.minimum(wid * PPW, NPTS - PPW)
        pltpu.sync_copy(xs_hbm.at[pl.ds(wbase, PPW)], xv.at[pl.ds(0, PPW)])
        pltpu.sync_copy(ys_hbm.at[pl.ds(wbase, PPW)], yv.at[pl.ds(0, PPW)])
        pltpu.sync_copy(base_hbm.at[pl.ds(wbase, PPW)], bv.at[pl.ds(0, PPW)])
        xv[pl.ds(PPW, PPH)] = xv[pl.ds(0, PPH)]
        yv[pl.ds(PPW, PPH)] = yv[pl.ds(0, PPH)]
        bv[pl.ds(PPW, PPH)] = bv[pl.ds(0, PPH)]

        def compute_idx(h, idx_v, wts_v):
            o = h * PPH
            x = xv[pl.ds(o, LANES)]
            y = yv[pl.ds(o, LANES)]
            bb = bv[pl.ds(o, LANES)]
            x = (x - PC_START[0]) / VOXEL_SIZE[0] / OUT_STRIDE
            y = (y - PC_START[1]) / VOXEL_SIZE[1] / OUT_STRIDE
            x = jnp.minimum(jnp.maximum(x, -4.0), 184.0)
            y = jnp.minimum(jnp.maximum(y, -4.0), 184.0)
            xi = x.astype(jnp.int32)
            yi = y.astype(jnp.int32)
            x0 = jnp.where(xi.astype(jnp.float32) > x, xi - 1, xi)
            y0 = jnp.where(yi.astype(jnp.float32) > y, yi - 1, yi)
            x0c = jnp.minimum(jnp.maximum(x0, 0), W - 1)
            x1c = jnp.minimum(jnp.maximum(x0 + 1, 0), W - 1)
            y0c = jnp.minimum(jnp.maximum(y0, 0), H - 1)
            y1c = jnp.minimum(jnp.maximum(y0 + 1, 0), H - 1)
            fx0 = x0c.astype(jnp.float32)
            fx1 = x1c.astype(jnp.float32)
            fy0 = y0c.astype(jnp.float32)
            fy1 = y1c.astype(jnp.float32)
            wa = (fx1 - x) * (fy1 - y)
            wb = (fx1 - x) * (y - fy0)
            wc = (x - fx0) * (fy1 - y)
            wd = (x - fx0) * (y - fy0)
            for l in range(LANES):
                wts_v[0 * PPH + l, :] = jnp.full((LANES,), wa[l])
                wts_v[1 * PPH + l, :] = jnp.full((LANES,), wb[l])
                wts_v[2 * PPH + l, :] = jnp.full((LANES,), wc[l])
                wts_v[3 * PPH + l, :] = jnp.full((LANES,), wd[l])
            row0 = bb + y0c * W
            row1 = bb + y1c * W
            idx_v[pl.ds(0 * PPH, LANES)] = row0 + x0c
            idx_v[pl.ds(1 * PPH, LANES)] = row1 + x0c
            idx_v[pl.ds(2 * PPH, LANES)] = row0 + x1c
            idx_v[pl.ds(3 * PPH, LANES)] = row1 + x1c

        def blend(rows_v, wts_v, out_v):
            for p0 in range(0, PPH, 8):
                wregs = [[wts_v[c * PPH + p0 + p, :] for p in range(8)]
                         for c in range(4)]

                @plsc.parallel_loop(0, C // LANES, unroll=2)
                def body(j, _wregs=wregs, _p0=p0):
                    s = pl.ds(j * LANES, LANES)
                    for p in range(8):
                        kk = _p0 + p
                        va = rows_v[kk, s]
                        vb = rows_v[PPH + kk, s]
                        vc = rows_v[2 * PPH + kk, s]
                        vd = rows_v[3 * PPH + kk, s]
                        out_v[kk, s] = (((va * _wregs[0][p] + vb * _wregs[1][p])
                                         + vc * _wregs[2][p]) + vd * _wregs[3][p])

        compute_idx(0, idx_a, wts_a)
        pltpu.async_copy(bev_hbm.at[idx_a], rows_a, gs_a)

        def pair_body(i, _):
            compute_idx(2 * i + 1, idx_b, wts_b)
            pltpu.async_copy(bev_hbm.at[idx_b], rows_b, gs_b)

            pltpu.make_async_copy(bev_hbm.at[idx_a], rows_a, gs_a).wait()

            @pl.when(i != 0)
            def _():
                pltpu.make_async_copy(out_a, out_hbm.at[pl.ds(wbase, PPH)], os_a).wait()
            blend(rows_a, wts_a, out_a)
            pltpu.async_copy(out_a, out_hbm.at[pl.ds(wbase + (2 * i) * PPH, PPH)], os_a)

            compute_idx(2 * i + 2, idx_a, wts_a)
            pltpu.async_copy(bev_hbm.at[idx_a], rows_a, gs_a)

            pltpu.make_async_copy(bev_hbm.at[idx_b], rows_b, gs_b).wait()

            @pl.when(i != 0)
            def _():
                pltpu.make_async_copy(out_b, out_hbm.at[pl.ds(wbase, PPH)], os_b).wait()
            blend(rows_b, wts_b, out_b)
            pltpu.async_copy(out_b, out_hbm.at[pl.ds(wbase + (2 * i + 1) * PPH, PPH)], os_b)
            return 0

        lax.fori_loop(0, NH // 2, pair_body, 0)

        pltpu.make_async_copy(bev_hbm.at[idx_a], rows_a, gs_a).wait()
        pltpu.make_async_copy(out_a, out_hbm.at[pl.ds(wbase, PPH)], os_a).wait()
        pltpu.make_async_copy(out_b, out_hbm.at[pl.ds(wbase, PPH)], os_b).wait()

    return k(bev2d, xs, ys, rowbase)


def kernel(bev_feature, batch_centers, num_point):
    del num_point
    cx = batch_centers[..., 0]
    cy = batch_centers[..., 1]
    order = (jnp.arange(SEC)[:, None] + SEC * jnp.arange(NSEC)[None, :]).reshape(-1)
    xs = cx[:, order].reshape(-1)
    ys = cy[:, order].reshape(-1)
    rowbase = jnp.repeat(jnp.arange(B, dtype=jnp.int32) * (H * W), N)
    bev2d = bev_feature.reshape(B * H * W, C)
    out = _sc_interp(bev2d, xs, ys, rowbase)
    return out.reshape(B, SEC, NSEC * C)
